# baseline (speedup 1.0000x reference)
import functools

import jax
import jax.numpy as jnp
from jax import lax
from jax.experimental import pallas as pl
from jax.experimental.pallas import tpu as pltpu
from jax.experimental.pallas import tpu_sc as plsc

_NUM_ENTITIES = 1_000_000
_NUM_RELATIONS = 1_000
_DIM = 64
_BATCH = 16_384
_MARGIN = 1.0
_EPSILON = 0.05

_ROWS2 = _NUM_ENTITIES // 2
_BR = 4000
_GRID = _ROWS2 // _BR


def _constraints_body(ent_ref, n_ref, p_ref, out_ref, acc_ref):
    i = pl.program_id(0)

    @pl.when(i == 0)
    def _():
        n = n_ref[...]
        p = p_ref[...]
        nn = jnp.sum(n * n, axis=1)
        npd = jnp.sum(n * p, axis=1)
        den = jnp.sum(p * p, axis=1)
        relc = jnp.sum(jnp.abs((npd * npd) / (nn * den)
                               - _NUM_RELATIONS * _EPSILON))
        acc_ref[0, 0] = relc

    x = ent_ref[...]
    s = x * x
    t = (jnp.abs(jnp.sum(s[:, :64], axis=1) - float(_NUM_ENTITIES))
         + jnp.abs(jnp.sum(s[:, 64:], axis=1) - float(_NUM_ENTITIES)))
    acc_ref[0, 0] += jnp.sum(t)

    @pl.when(i == _GRID - 1)
    def _():
        out_ref[0, 0] = acc_ref[0, 0]


def _constraints(ent2, normal_emb, proj_relation_emb):
    return pl.pallas_call(
        _constraints_body,
        grid=(_GRID,),
        in_specs=[
            pl.BlockSpec((_BR, 128), lambda i: (i, 0)),
            pl.BlockSpec((_NUM_RELATIONS, _DIM), lambda i: (0, 0)),
            pl.BlockSpec((_NUM_RELATIONS, _DIM), lambda i: (0, 0)),
        ],
        out_specs=pl.BlockSpec(memory_space=pltpu.SMEM),
        out_shape=jax.ShapeDtypeStruct((1, 1), jnp.float32),
        scratch_shapes=[pltpu.SMEM((1, 1), jnp.float32)],
        compiler_params=pltpu.CompilerParams(
            dimension_semantics=("arbitrary",)),
    )(ent2, normal_emb, proj_relation_emb)


_NW = 32
_NPER = _BATCH // _NW
_CH = 128
_NCH = _NPER // _CH
_NG = _CH // 16

_mesh = plsc.VectorSubcoreMesh(core_axis_name="c", subcore_axis_name="s")


@functools.partial(
    pl.kernel,
    out_type=jax.ShapeDtypeStruct((_NW, 16), jnp.float32),
    mesh=_mesh,
    scratch_types=[
        pltpu.VMEM((_CH,), jnp.int32),
        pltpu.VMEM((_CH,), jnp.int32),
        pltpu.VMEM((_CH,), jnp.int32),
        pltpu.VMEM((_CH, _DIM), jnp.float32),
        pltpu.VMEM((_CH, _DIM), jnp.float32),
        pltpu.VMEM((_CH, _DIM), jnp.float32),
        pltpu.VMEM((_CH, _DIM), jnp.float32),
        pltpu.VMEM((_CH,), jnp.float32),
        pltpu.VMEM((16,), jnp.float32),
        pltpu.SemaphoreType.DMA,
    ],
    compiler_params=pltpu.CompilerParams(needs_layout_passes=False,
                                         use_tc_tiling_on_sc=False),
)
def _margin_kernel(ph_h, pt_h, pr_h, nh_h, nt_h, nr_h, ent_h, rel_h, nrm_h,
                   out_h, ib1, ib2, ib3, hbuf, tbuf, rbuf, nbuf, sbuf, obuf,
                   sem):
    cid = lax.axis_index("c")
    sid = lax.axis_index("s")
    wid = sid * 2 + cid
    base = wid * _NPER
    iot = lax.iota(jnp.int32, 16)
    zero = jnp.zeros(16, jnp.float32)

    def gather_rows(hsrc, tsrc, rsrc, off):
        pltpu.sync_copy(hsrc.at[pl.ds(off, _CH)], ib1)
        pltpu.sync_copy(tsrc.at[pl.ds(off, _CH)], ib2)
        pltpu.sync_copy(rsrc.at[pl.ds(off, _CH)], ib3)
        c1 = pltpu.async_copy(ent_h.at[ib1], hbuf, sem)
        c2 = pltpu.async_copy(ent_h.at[ib2], tbuf, sem)
        c3 = pltpu.async_copy(rel_h.at[ib3], rbuf, sem)
        c4 = pltpu.async_copy(nrm_h.at[ib3], nbuf, sem)
        c1.wait()
        c2.wait()
        c3.wait()
        c4.wait()

    def group_score(g):
        rvec = g * 16 + iot

        def pass1(d, carry):
            nh, nt, nn = carry
            cvec = jnp.full((16,), d, jnp.int32)
            hv = plsc.load_gather(hbuf, [rvec, cvec])
            tv = plsc.load_gather(tbuf, [rvec, cvec])
            rv = plsc.load_gather(rbuf, [rvec, cvec])
            nv = plsc.load_gather(nbuf, [rvec, cvec])
            plsc.store_scatter(hbuf, [rvec, cvec], hv - tv + rv)
            return (nh + nv * hv, nt + nv * tv, nn + nv * nv)

        nh, nt, nn = lax.fori_loop(0, _DIM, pass1, (zero, zero, zero))
        a = (nh - nt) / nn

        def pass2(d, acc):
            cvec = jnp.full((16,), d, jnp.int32)
            uv = plsc.load_gather(hbuf, [rvec, cvec])
            nv = plsc.load_gather(nbuf, [rvec, cvec])
            return acc + jnp.abs(uv - a * nv)

        sm = lax.fori_loop(0, _DIM, pass2, zero)
        return -(sm * sm)

    macc = zero
    for ch in range(_NCH):
        off = base + ch * _CH

        gather_rows(ph_h, pt_h, pr_h, off)

        def pos_group(g, _):
            sbuf[pl.ds(g * 16, 16)] = group_score(g)
            return 0

        lax.fori_loop(0, _NG, pos_group, 0)

        gather_rows(nh_h, nt_h, nr_h, off)

        def neg_group(g, m):
            sn = group_score(g)
            sp = sbuf[pl.ds(g * 16, 16)]
            return m + jnp.maximum(sp - sn + _MARGIN, 0.0)

        macc = lax.fori_loop(0, _NG, neg_group, macc)

    obuf[...] = macc
    pltpu.sync_copy(obuf, out_h.at[wid])


def kernel(pos_heads, pos_rels, pos_tails, neg_heads, neg_rels, neg_tails,
           entity_emb, relation_emb, proj_relation_emb, normal_emb, c):
    ent2 = entity_emb.reshape(_ROWS2, 2 * _DIM)
    cons = _constraints(ent2, normal_emb, proj_relation_emb)[0, 0]
    parts = _margin_kernel(pos_heads, pos_tails, pos_rels,
                           neg_heads, neg_tails, neg_rels,
                           entity_emb, relation_emb, normal_emb)
    margin = jnp.sum(parts)
    return margin + c * cons
